# Initial kernel scaffold; baseline (speedup 1.0000x reference)
import jax
import jax.numpy as jnp
from jax.experimental import pallas as pl


def kernel(x, w1s, w2s):
    raise NotImplementedError("write your pallas kernel here")


# speedup vs baseline: 42.5990x; 42.5990x over previous
import functools

import jax
import jax.numpy as jnp
import numpy as np
from jax import lax
from jax.experimental import pallas as pl
from jax.experimental.pallas import tpu as pltpu
from jax.experimental.pallas import tpu_sc as plsc

W = 1024
NLEV = 13
NN = 8191
B = 8192
NC = 2
NS = 16
NW = NC * NS
TPW = B // NW
CH = 32
NCHUNK = TPW // CH
NQ = 8
PSZ = NQ * W

_INV_SQRT2 = float(1.0 / np.sqrt(2.0))

_MESH = plsc.VectorSubcoreMesh(core_axis_name="c", subcore_axis_name="s")


def _wid():
    return lax.axis_index("s") * NC + lax.axis_index("c")


def _gelu(v):
    z = v * _INV_SQRT2
    s = jnp.sign(z)
    a = jnp.abs(z)
    t = 1.0 / (1.0 + 0.3275911 * a)
    poly = ((((1.061405429 * t - 1.453152027) * t + 1.421413741) * t
             - 0.284496736) * t + 0.254829592) * t
    erf = s * (1.0 - poly * jnp.exp(-a * a))
    return 0.5 * v * (1.0 + erf)


def _zero_acc(accb):
    def zj(j, _):
        for r8 in range(NQ):
            accb[pl.ds(r8 * W + j * 16, 16)] = jnp.zeros((16,), jnp.float32)
        return 0
    lax.fori_loop(0, W // 16, zj, 0)


def _accumulate_chunk(xb, gb, accb):
    def rb_body(rb, _):
        base = rb * 64
        acc = [accb[pl.ds(r8 * W + base + 16 * k, 16)]
               for r8 in range(NQ) for k in range(4)]

        def u8_body(u8, acc):
            acc = list(acc)
            for v in range(8):
                tloc = u8 * 8 + v
                for k in range(4):
                    sl = pl.ds(base + 16 * k, 16)
                    acc[v * 4 + k] = acc[v * 4 + k] + xb[tloc, sl] * gb[tloc, sl]
            return tuple(acc)

        acc = lax.fori_loop(0, CH // 8, u8_body, tuple(acc))
        for r8 in range(NQ):
            for k in range(4):
                accb[pl.ds(r8 * W + base + 16 * k, 16)] = acc[r8 * 4 + k]
        return 0
    lax.fori_loop(0, W // 64, rb_body, 0)


def _stage_slices(part_in, redb, sem, wid):
    off = (wid // 4) * W + (wid % 4) * 256
    copies = [
        pltpu.async_copy(part_in.at[pl.ds(v * PSZ + off, TPW)],
                         redb.at[pl.ds(v * TPW, TPW)], sem)
        for v in range(NW)
    ]
    for c in copies:
        c.wait()


def _level_tail(x_hbm, w1_hbm, part_out, nodes_vec_ref, xb, gb, accb, idxb,
                sem, wid):
    t0 = wid * TPW
    _zero_acc(accb)

    def chunk(ci, _):
        tb = t0 + ci * CH
        pltpu.sync_copy(x_hbm.at[pl.ds(tb, CH)], xb)
        for half in range(CH // 16):
            idxb[pl.ds(half * 16, 16)] = nodes_vec_ref[
                pl.ds(ci * CH + half * 16, 16)]
        pltpu.async_copy(w1_hbm.at[idxb], gb, sem).wait()
        _accumulate_chunk(xb, gb, accb)
        return 0

    lax.fori_loop(0, NCHUNK, chunk, 0)
    pltpu.sync_copy(accb, part_out.at[pl.ds(wid * PSZ, PSZ)])


@functools.partial(
    pl.kernel,
    out_type=jax.ShapeDtypeStruct((NW * PSZ,), jnp.float32),
    mesh=_MESH,
    scratch_types=[
        pltpu.VMEM((CH, W), jnp.float32),
        pltpu.VMEM((CH, W), jnp.float32),
        pltpu.VMEM((PSZ,), jnp.float32),
        pltpu.VMEM((TPW,), jnp.int32),
        pltpu.VMEM((CH,), jnp.int32),
        pltpu.SemaphoreType.DMA,
    ],
)
def _k_first(x_hbm, w1_hbm, part_out, xb, gb, accb, nodesb, idxb, sem):
    wid = _wid()
    for j in range(TPW // 16):
        nodesb[pl.ds(j * 16, 16)] = jnp.zeros((16,), jnp.int32)
    _level_tail(x_hbm, w1_hbm, part_out, nodesb, xb, gb, accb, idxb, sem, wid)


@functools.partial(
    pl.kernel,
    out_type=[
        jax.ShapeDtypeStruct((B,), jnp.float32),
        jax.ShapeDtypeStruct((B,), jnp.int32),
        jax.ShapeDtypeStruct((NW * PSZ,), jnp.float32),
    ],
    mesh=_MESH,
    scratch_types=[
        pltpu.VMEM((CH, W), jnp.float32),
        pltpu.VMEM((CH, W), jnp.float32),
        pltpu.VMEM((PSZ,), jnp.float32),
        pltpu.VMEM((NW * TPW,), jnp.float32),
        pltpu.VMEM((TPW,), jnp.int32),
        pltpu.VMEM((TPW,), jnp.int32),
        pltpu.VMEM((TPW,), jnp.float32),
        pltpu.VMEM((CH,), jnp.int32),
        pltpu.SemaphoreType.DMA,
    ],
)
def _k_mid(x_hbm, w1_hbm, part_in, nodes_in, h_out, nodes_out, part_out,
           xb, gb, accb, redb, nin_b, nout_b, hb, idxb, sem):
    wid = _wid()
    t0 = wid * TPW
    _stage_slices(part_in, redb, sem, wid)
    pltpu.sync_copy(nodes_in.at[pl.ds(t0, TPW)], nin_b)

    def sblk(j, _):
        sl = pl.ds(j * 16, 16)
        l16 = redb[pl.ds(j * 16, 16)]
        for v in range(1, NW):
            l16 = l16 + redb[pl.ds(v * TPW + j * 16, 16)]
        hb[sl] = _gelu(l16)
        choice = jnp.minimum(1.0, jnp.maximum(0.0, 1.0 + l16 * 1e30 * 1e30))
        nout_b[sl] = nin_b[sl] * 2 + choice.astype(jnp.int32) + 1
        return 0

    lax.fori_loop(0, TPW // 16, sblk, 0)
    pltpu.sync_copy(hb, h_out.at[pl.ds(t0, TPW)])
    pltpu.sync_copy(nout_b, nodes_out.at[pl.ds(t0, TPW)])
    _level_tail(x_hbm, w1_hbm, part_out, nout_b, xb, gb, accb, idxb, sem, wid)


@functools.partial(
    pl.kernel,
    out_type=jax.ShapeDtypeStruct((B,), jnp.float32),
    mesh=_MESH,
    scratch_types=[
        pltpu.VMEM((NW * TPW,), jnp.float32),
        pltpu.VMEM((TPW,), jnp.float32),
        pltpu.SemaphoreType.DMA,
    ],
)
def _k_last(part_in, h_out, redb, hb, sem):
    wid = _wid()
    t0 = wid * TPW
    _stage_slices(part_in, redb, sem, wid)

    def sblk(j, _):
        sl = pl.ds(j * 16, 16)
        l16 = redb[pl.ds(j * 16, 16)]
        for v in range(1, NW):
            l16 = l16 + redb[pl.ds(v * TPW + j * 16, 16)]
        hb[sl] = _gelu(l16)
        return 0

    lax.fori_loop(0, TPW // 16, sblk, 0)
    pltpu.sync_copy(hb, h_out.at[pl.ds(t0, TPW)])


@functools.partial(
    pl.kernel,
    out_type=jax.ShapeDtypeStruct((B, W), jnp.float32),
    mesh=_MESH,
    scratch_types=[
        pltpu.VMEM((NLEV * TPW,), jnp.float32),
        pltpu.VMEM((NLEV * TPW,), jnp.int32),
        pltpu.VMEM((16,), jnp.int32),
        pltpu.VMEM((16, W), jnp.float32),
        pltpu.VMEM((16, W), jnp.float32),
        pltpu.SemaphoreType.DMA,
    ],
)
def _k_ph2(h_flat, n_flat, w2_hbm, out_hbm, hstg, nstg, idxb, g2b, accb, sem):
    wid = _wid()
    b0 = wid * TPW

    for m in range(NLEV):
        pltpu.sync_copy(h_flat.at[pl.ds(b0 + B * m, TPW)],
                        hstg.at[pl.ds(m * TPW, TPW)])
        pltpu.sync_copy(n_flat.at[pl.ds(b0 + B * m, TPW)],
                        nstg.at[pl.ds(m * TPW, TPW)])

    def bc_body(bc, _):
        def zj(j, _):
            for u in range(16):
                accb[u, pl.ds(j * 16, 16)] = jnp.zeros((16,), jnp.float32)
            return 0
        lax.fori_loop(0, W // 16, zj, 0)

        for m in range(NLEV):
            idxb[...] = nstg[pl.ds(m * TPW + bc * 16, 16)]
            pltpu.async_copy(w2_hbm.at[idxb], g2b, sem).wait()
            hv = hstg[pl.ds(m * TPW + bc * 16, 16)]
            for u in range(16):
                hsp = jnp.zeros((16,), jnp.float32) + hv[u]

                def jj(j, _, u=u, hsp=hsp):
                    sl = pl.ds(j * 16, 16)
                    accb[u, sl] = accb[u, sl] + hsp * g2b[u, sl]
                    return 0

                lax.fori_loop(0, W // 16, jj, 0)

        pltpu.sync_copy(accb, out_hbm.at[pl.ds(b0 + bc * 16, 16)])
        return 0

    lax.fori_loop(0, TPW // 16, bc_body, 0)


def kernel(x, w1s, w2s):
    w1 = w1s.reshape(NN, W)
    part = _k_first(x, w1)
    nodes_cols = [jnp.zeros((B,), jnp.int32)]
    h_cols = []
    for _ in range(NLEV - 1):
        h_prev, n_next, part = _k_mid(x, w1, part, nodes_cols[-1])
        h_cols.append(h_prev)
        nodes_cols.append(n_next)
    h_cols.append(_k_last(part))
    h_flat = jnp.stack(h_cols, axis=1).reshape(-1)
    n_flat = jnp.stack(nodes_cols, axis=1).reshape(-1)
    return _k_ph2(h_flat, n_flat, w2s)
